# baseline (speedup 1.0000x reference)
import functools

import jax
import jax.numpy as jnp
from jax import lax
from jax.experimental import pallas as pl
from jax.experimental.pallas import tpu as pltpu
from jax.experimental.pallas import tpu_sc as plsc

NC, NS = 2, 16
NW = NC * NS
B2 = 8192
DIM = 128
CHUNK = 128
PER_W = B2 // NW
NCHUNK = PER_W // CHUNK
NGATHER = 12
NBUF = 4


def _sc_gather_all(tables, idxs):
    mesh = plsc.VectorSubcoreMesh(core_axis_name="c", subcore_axis_name="s")
    out_type = [jax.ShapeDtypeStruct((B2, DIM), jnp.float32)] * NGATHER
    nslots = NGATHER * NCHUNK

    @functools.partial(
        pl.kernel,
        mesh=mesh,
        out_type=out_type,
        scratch_types=(
            [pltpu.VMEM((PER_W,), jnp.int32)] * NGATHER
            + [pltpu.VMEM((CHUNK, DIM), jnp.float32)] * NBUF
            + [pltpu.SemaphoreType.DMA] * (1 + 2 * NBUF)
        ),
    )
    def k(*refs):
        t_refs = refs[:NGATHER]
        i_refs = refs[NGATHER:2 * NGATHER]
        o_refs = refs[2 * NGATHER:3 * NGATHER]
        idx_v = refs[3 * NGATHER:4 * NGATHER]
        rows_v = refs[4 * NGATHER:4 * NGATHER + NBUF]
        sem_i = refs[4 * NGATHER + NBUF]
        sem_g = refs[4 * NGATHER + NBUF + 1:4 * NGATHER + 2 * NBUF + 1]
        sem_w = refs[4 * NGATHER + 2 * NBUF + 1:4 * NGATHER + 3 * NBUF + 1]
        wid = lax.axis_index("s") * NC + lax.axis_index("c")
        base0 = wid * PER_W

        pf = [pltpu.make_async_copy(i_refs[g].at[pl.ds(base0, PER_W)],
                                    idx_v[g], sem_i) for g in range(NGATHER)]
        for d in pf:
            d.start()
        for d in pf:
            d.wait()

        def slot(kk):
            g, c = kk // NCHUNK, kk % NCHUNK
            return g, c * CHUNK

        gd = [None] * NBUF
        wd = [None] * NBUF
        for kk in range(nslots):
            s = kk % NBUF
            if wd[s] is not None:
                wd[s].wait()
                wd[s] = None
            g, off = slot(kk)
            gd[s] = pltpu.make_async_copy(
                t_refs[g].at[idx_v[g].at[pl.ds(off, CHUNK)]],
                rows_v[s], sem_g[s])
            gd[s].start()
            if kk >= NBUF - 1:
                p = (kk - (NBUF - 1)) % NBUF
                gd[p].wait()
                pg, poff = slot(kk - (NBUF - 1))
                wd[p] = pltpu.make_async_copy(
                    rows_v[p], o_refs[pg].at[pl.ds(base0 + poff, CHUNK)],
                    sem_w[p])
                wd[p].start()
        for kk in range(max(0, nslots - (NBUF - 1)), nslots):
            s = kk % NBUF
            gd[s].wait()
            pg, poff = slot(kk)
            wd[s] = pltpu.make_async_copy(
                rows_v[s], o_refs[pg].at[pl.ds(base0 + poff, CHUNK)], sem_w[s])
            wd[s].start()
        for s in range(NBUF):
            if wd[s] is not None:
                wd[s].wait()

    return k(*tables, *idxs)


def _tc_score(g):
    blk = 512

    def body(h, sh0, sh1, sh2, t, st0, st1, st2, r, sq0, sq1, sq2, o):
        def nrm(x):
            n = jnp.sqrt(jnp.sum(x * x, axis=1, keepdims=True))
            return x / jnp.maximum(n, 1e-12)

        hv = nrm(h[...]) + nrm(sh0[...]) + nrm(sh1[...]) + nrm(sh2[...])
        tv = nrm(t[...]) + nrm(st0[...]) + nrm(st1[...]) + nrm(st2[...])
        rv = nrm(r[...]) + nrm(sq0[...]) + nrm(sq1[...]) + nrm(sq2[...])
        d = nrm(hv) + nrm(rv) - nrm(tv)
        o[...] = jnp.sqrt(jnp.sum(d * d, axis=1, keepdims=True))

    in_spec = pl.BlockSpec((blk, DIM), lambda i: (i, 0))
    out_spec = pl.BlockSpec((blk, 1), lambda i: (i, 0))
    return pl.pallas_call(
        body,
        grid=(B2 // blk,),
        in_specs=[in_spec] * NGATHER,
        out_specs=out_spec,
        out_shape=jax.ShapeDtypeStruct((B2, 1), jnp.float32),
    )(*g)


def kernel(ents, rels_tab, se0, se1, se2, sr0, sr1, sr2,
           heads, rels, tails, sources,
           heads_bad, rels_bad, tails_bad, sources_bad):
    ah = jnp.concatenate([heads, heads_bad]).astype(jnp.int32)
    ar = jnp.concatenate([rels, rels_bad]).astype(jnp.int32)
    at = jnp.concatenate([tails, tails_bad]).astype(jnp.int32)
    asrc = jnp.concatenate([sources, sources_bad])
    z = jnp.zeros((), jnp.int32)
    masks = [asrc == (j + 2) for j in range(3)]

    tables = ([ents] + [se0, se1, se2]
              + [ents] + [se0, se1, se2]
              + [rels_tab] + [sr0, sr1, sr2])
    idxs = ([ah] + [jnp.where(m, ah, z) for m in masks]
            + [at] + [jnp.where(m, at, z) for m in masks]
            + [ar] + [jnp.where(m, ar, z) for m in masks])

    g = _sc_gather_all(tables, idxs)
    s = _tc_score(g)[:, 0]
    return (s[:4096], s[4096:])


# speedup vs baseline: 7.6391x; 7.6391x over previous
import dataclasses
import functools

import jax
import jax.numpy as jnp
from jax import lax
from jax.experimental import pallas as pl
from jax.experimental.pallas import tpu as pltpu
from jax.experimental.pallas import tpu_sc as plsc

NC, NS = 2, 16
NW = NC * NS
B2 = 8192
DIM = 128
CHUNK = 128
PER_W = B2 // NW
NCHUNK = PER_W // CHUNK
NBUF = 4


def _sc_gather_all(tables, idxs, gsel, bounds):
    mesh = plsc.VectorSubcoreMesh(core_axis_name="c", subcore_axis_name="s")
    ng = len(tables)
    out_type = [jax.ShapeDtypeStruct((B2, DIM), jnp.float32)] * ng
    cp = pltpu.CompilerParams()
    if "needs_layout_passes" in pltpu.CompilerParams.__dataclass_fields__:
        cp = dataclasses.replace(cp, needs_layout_passes=False)

    @functools.partial(
        pl.kernel,
        mesh=mesh,
        out_type=out_type,
        compiler_params=cp,
        scratch_types=(
            [pltpu.VMEM((PER_W,), jnp.int32)] * 3
            + [pltpu.VMEM((CHUNK, DIM), jnp.float32)] * NBUF
            + [pltpu.VMEM((16,), jnp.int32)]
            + [pltpu.SemaphoreType.DMA] * (1 + 2 * NBUF)
        ),
    )
    def k(*refs):
        t_refs = refs[:ng]
        i_refs = refs[ng:ng + 3]
        b_ref = refs[ng + 3]
        o_refs = refs[ng + 4:2 * ng + 4]
        base = 2 * ng + 4
        idx_v = refs[base:base + 3]
        rows_v = refs[base + 3:base + 3 + NBUF]
        bnd_s = refs[base + 3 + NBUF]
        sem_i = refs[base + 4 + NBUF]
        sem_g = refs[base + 5 + NBUF:base + 5 + 2 * NBUF]
        sem_w = refs[base + 5 + 2 * NBUF:base + 5 + 3 * NBUF]
        wid = lax.axis_index("s") * NC + lax.axis_index("c")
        base0 = wid * PER_W

        pltpu.sync_copy(b_ref, bnd_s)
        pf = [pltpu.make_async_copy(i_refs[a].at[pl.ds(base0, PER_W)],
                                    idx_v[a], sem_i) for a in range(3)]
        for d in pf:
            d.start()
        for d in pf:
            d.wait()

        bvec = bnd_s[...]
        lane = jnp.arange(16, dtype=jnp.int32)
        bs = [jnp.max(jnp.where(lane == j, bvec, 0)) for j in range(4)]

        slots = []
        for g in range(ng):
            a, j = gsel[g]
            for c in range(NCHUNK):
                slots.append((g, a, c * CHUNK, j))

        def cond_of(j, off):
            if j is None:
                return None
            p = base0 + off
            return jnp.logical_and(p < bs[j + 1], p + CHUNK > bs[j])

        def guarded(cond, fn):
            if cond is None:
                fn()
            else:
                pl.when(cond)(fn)

        conds = [cond_of(j, off) for (_, _, off, j) in slots]
        gd = [None] * NBUF
        wd = [None] * NBUF

        def issue_writeback(kk):
            s = kk % NBUF
            g, _, off, _ = slots[kk]
            d, c = gd[s]
            guarded(c, d.wait)
            w = pltpu.make_async_copy(
                rows_v[s], o_refs[g].at[pl.ds(base0 + off, CHUNK)], sem_w[s])
            guarded(c, w.start)
            wd[s] = (w, c)

        for kk in range(len(slots)):
            s = kk % NBUF
            if wd[s] is not None:
                d, c = wd[s]
                guarded(c, d.wait)
                wd[s] = None
            g, a, off, j = slots[kk]
            d = pltpu.make_async_copy(
                t_refs[g].at[idx_v[a].at[pl.ds(off, CHUNK)]],
                rows_v[s], sem_g[s])
            guarded(conds[kk], d.start)
            gd[s] = (d, conds[kk])
            if kk >= NBUF - 1:
                issue_writeback(kk - (NBUF - 1))
        for kk in range(max(0, len(slots) - (NBUF - 1)), len(slots)):
            issue_writeback(kk)
        for s in range(NBUF):
            if wd[s] is not None:
                d, c = wd[s]
                guarded(c, d.wait)

    return k(*tables, *idxs, bounds)


def _tc_score(g, key_p):
    blk = 512

    def body(kp, h, sh0, sh1, sh2, t, st0, st1, st2, r, sq0, sq1, sq2, o):
        def nrm(x):
            n = jnp.sqrt(jnp.sum(x * x, axis=1, keepdims=True))
            return x / jnp.maximum(n, 1e-12)

        kk = kp[...]

        def sel(j, x):
            return jnp.where(kk == j, nrm(x[...]), 0.0)

        hv = nrm(h[...]) + sel(0, sh0) + sel(1, sh1) + sel(2, sh2)
        tv = nrm(t[...]) + sel(0, st0) + sel(1, st1) + sel(2, st2)
        rv = nrm(r[...]) + sel(0, sq0) + sel(1, sq1) + sel(2, sq2)
        d = nrm(hv) + nrm(rv) - nrm(tv)
        o[...] = jnp.sqrt(jnp.sum(d * d, axis=1, keepdims=True))

    in_specs = ([pl.BlockSpec((blk, 1), lambda i: (i, 0))]
                + [pl.BlockSpec((blk, DIM), lambda i: (i, 0))] * 12)
    out_spec = pl.BlockSpec((blk, 1), lambda i: (i, 0))
    return pl.pallas_call(
        body,
        grid=(B2 // blk,),
        in_specs=in_specs,
        out_specs=out_spec,
        out_shape=jax.ShapeDtypeStruct((B2, 1), jnp.float32),
    )(key_p, *g)


def kernel(ents, rels_tab, se0, se1, se2, sr0, sr1, sr2,
           heads, rels, tails, sources,
           heads_bad, rels_bad, tails_bad, sources_bad):
    ah = jnp.concatenate([heads, heads_bad]).astype(jnp.int32)
    ar = jnp.concatenate([rels, rels_bad]).astype(jnp.int32)
    at = jnp.concatenate([tails, tails_bad]).astype(jnp.int32)
    asrc = jnp.concatenate([sources, sources_bad]).astype(jnp.int32)

    key = jnp.where((asrc >= 2) & (asrc <= 4), asrc - 2, 3)
    iot = jnp.arange(B2, dtype=jnp.int32)
    csum = [jnp.cumsum((key == j).astype(jnp.int32)) for j in range(4)]
    n = [c[-1] for c in csum]
    b0 = jnp.int32(0)
    b1, b2, b3 = n[0], n[0] + n[1], n[0] + n[1] + n[2]
    pos = jnp.where(key == 0, csum[0] - 1,
          jnp.where(key == 1, b1 + csum[1] - 1,
          jnp.where(key == 2, b2 + csum[2] - 1, b3 + csum[3] - 1)))
    perm = jnp.zeros((B2,), jnp.int32).at[pos].set(iot)
    bounds = jnp.stack([b0, b1, b2, b3] + [b0] * 12).astype(jnp.int32)

    ah_p, at_p, ar_p = ah[perm], at[perm], ar[perm]
    key_p = key[perm].reshape(B2, 1)

    tables = [ents, se0, se1, se2, ents, se0, se1, se2,
              rels_tab, sr0, sr1, sr2]
    gsel = [(0, None), (0, 0), (0, 1), (0, 2),
            (1, None), (1, 0), (1, 1), (1, 2),
            (2, None), (2, 0), (2, 1), (2, 2)]

    g = _sc_gather_all(tables, [ah_p, at_p, ar_p], gsel, bounds)
    s_perm = _tc_score(g, key_p)[:, 0]
    s = s_perm[pos]
    return (s[:4096], s[4096:])


# speedup vs baseline: 9.5248x; 1.2468x over previous
import dataclasses
import functools

import jax
import jax.numpy as jnp
from jax import lax
from jax.experimental import pallas as pl
from jax.experimental.pallas import tpu as pltpu
from jax.experimental.pallas import tpu_sc as plsc

NC, NS = 2, 16
NW = NC * NS
B2 = 8192
DIM = 128
CHUNK = 128
PER_W = B2 // NW
NCHUNK = PER_W // CHUNK
NBUF = 4


def _sc_gather_all(tables, idxs, gsel, bounds, perm):
    mesh = plsc.VectorSubcoreMesh(core_axis_name="c", subcore_axis_name="s")
    ng = len(tables)
    out_type = [jax.ShapeDtypeStruct((B2, DIM), jnp.float32)] * ng
    cp = pltpu.CompilerParams()
    if "needs_layout_passes" in pltpu.CompilerParams.__dataclass_fields__:
        cp = dataclasses.replace(cp, needs_layout_passes=False)

    @functools.partial(
        pl.kernel,
        mesh=mesh,
        out_type=out_type,
        compiler_params=cp,
        scratch_types=(
            [pltpu.VMEM((PER_W,), jnp.int32)] * 4
            + [pltpu.VMEM((CHUNK, DIM), jnp.float32)] * NBUF
            + [pltpu.VMEM((16,), jnp.int32)]
            + [pltpu.SemaphoreType.DMA] * (1 + 2 * NBUF)
        ),
    )
    def k(*refs):
        t_refs = refs[:ng]
        i_refs = refs[ng:ng + 3]
        b_ref = refs[ng + 3]
        p_ref = refs[ng + 4]
        o_refs = refs[ng + 5:2 * ng + 5]
        base = 2 * ng + 5
        idx_v = refs[base:base + 3]
        perm_v = refs[base + 3]
        rows_v = refs[base + 4:base + 4 + NBUF]
        bnd_s = refs[base + 4 + NBUF]
        sem_i = refs[base + 5 + NBUF]
        sem_g = refs[base + 6 + NBUF:base + 6 + 2 * NBUF]
        sem_w = refs[base + 6 + 2 * NBUF:base + 6 + 3 * NBUF]
        wid = lax.axis_index("s") * NC + lax.axis_index("c")
        base0 = wid * PER_W

        pltpu.sync_copy(b_ref, bnd_s)
        pltpu.sync_copy(p_ref.at[pl.ds(base0, PER_W)], perm_v)
        pf = [pltpu.make_async_copy(i_refs[a].at[perm_v], idx_v[a], sem_i)
              for a in range(3)]
        for d in pf:
            d.start()
        for d in pf:
            d.wait()

        bvec = bnd_s[...]
        lane = jnp.arange(16, dtype=jnp.int32)
        bs = [jnp.max(jnp.where(lane == j, bvec, 0)) for j in range(4)]

        slots = []
        for g in range(ng):
            a, j = gsel[g]
            for c in range(NCHUNK):
                slots.append((g, a, c * CHUNK, j))

        def cond_of(j, off):
            if j is None:
                return None
            p = base0 + off
            return jnp.logical_and(p < bs[j + 1], p + CHUNK > bs[j])

        def guarded(cond, fn):
            if cond is None:
                fn()
            else:
                pl.when(cond)(fn)

        conds = [cond_of(j, off) for (_, _, off, j) in slots]
        gd = [None] * NBUF
        wd = [None] * NBUF

        def issue_writeback(kk):
            s = kk % NBUF
            g, _, off, _ = slots[kk]
            d, c = gd[s]
            guarded(c, d.wait)
            w = pltpu.make_async_copy(
                rows_v[s], o_refs[g].at[pl.ds(base0 + off, CHUNK)], sem_w[s])
            guarded(c, w.start)
            wd[s] = (w, c)

        for kk in range(len(slots)):
            s = kk % NBUF
            if wd[s] is not None:
                d, c = wd[s]
                guarded(c, d.wait)
                wd[s] = None
            g, a, off, j = slots[kk]
            d = pltpu.make_async_copy(
                t_refs[g].at[idx_v[a].at[pl.ds(off, CHUNK)]],
                rows_v[s], sem_g[s])
            guarded(conds[kk], d.start)
            gd[s] = (d, conds[kk])
            if kk >= NBUF - 1:
                issue_writeback(kk - (NBUF - 1))
        for kk in range(max(0, len(slots) - (NBUF - 1)), len(slots)):
            issue_writeback(kk)
        for s in range(NBUF):
            if wd[s] is not None:
                d, c = wd[s]
                guarded(c, d.wait)

    return k(*tables, *idxs, bounds, perm)


def _tc_score(g, bounds_tc):
    blk = 512

    def body(bnd, h, sh0, sh1, sh2, t, st0, st1, st2, r, sq0, sq1, sq2, o):
        def nrm(x):
            s = jnp.sum(x * x, axis=1, keepdims=True)
            return x * lax.rsqrt(jnp.maximum(s, 1e-24))

        rid = (pl.program_id(0) * blk
               + lax.broadcasted_iota(jnp.int32, (blk, 1), 0))

        def sel(j, x):
            in_grp = jnp.logical_and(rid >= bnd[0, j], rid < bnd[0, j + 1])
            return jnp.where(in_grp, nrm(x[...]), 0.0)

        hv = nrm(h[...]) + sel(0, sh0) + sel(1, sh1) + sel(2, sh2)
        tv = nrm(t[...]) + sel(0, st0) + sel(1, st1) + sel(2, st2)
        rv = nrm(r[...]) + sel(0, sq0) + sel(1, sq1) + sel(2, sq2)
        d = nrm(hv) + nrm(rv) - nrm(tv)
        o[...] = jnp.sqrt(jnp.sum(d * d, axis=1, keepdims=True))

    in_specs = ([pl.BlockSpec((1, 128), lambda i: (0, 0))]
                + [pl.BlockSpec((blk, DIM), lambda i: (i, 0))] * 12)
    out_spec = pl.BlockSpec((blk, 1), lambda i: (i, 0))
    return pl.pallas_call(
        body,
        grid=(B2 // blk,),
        in_specs=in_specs,
        out_specs=out_spec,
        out_shape=jax.ShapeDtypeStruct((B2, 1), jnp.float32),
    )(bounds_tc, *g)


def kernel(ents, rels_tab, se0, se1, se2, sr0, sr1, sr2,
           heads, rels, tails, sources,
           heads_bad, rels_bad, tails_bad, sources_bad):
    ah = jnp.concatenate([heads, heads_bad]).astype(jnp.int32)
    ar = jnp.concatenate([rels, rels_bad]).astype(jnp.int32)
    at = jnp.concatenate([tails, tails_bad]).astype(jnp.int32)
    asrc = jnp.concatenate([sources, sources_bad]).astype(jnp.int32)

    key = jnp.where((asrc >= 2) & (asrc <= 4), asrc - 2, 3)
    iot = jnp.arange(B2, dtype=jnp.int32)
    csum = [jnp.cumsum((key == j).astype(jnp.int32)) for j in range(4)]
    n = [c[-1] for c in csum]
    b0 = jnp.int32(0)
    b1, b2, b3 = n[0], n[0] + n[1], n[0] + n[1] + n[2]
    pos = jnp.where(key == 0, csum[0] - 1,
          jnp.where(key == 1, b1 + csum[1] - 1,
          jnp.where(key == 2, b2 + csum[2] - 1, b3 + csum[3] - 1)))
    perm = jnp.zeros((B2,), jnp.int32).at[pos].set(iot)
    bounds = jnp.stack([b0, b1, b2, b3] + [b0] * 12).astype(jnp.int32)
    bounds_tc = (jnp.zeros((1, 128), jnp.int32)
                 .at[0, 1].set(b1).at[0, 2].set(b2).at[0, 3].set(b3))

    tables = [ents, se0, se1, se2, ents, se0, se1, se2,
              rels_tab, sr0, sr1, sr2]
    gsel = [(0, None), (0, 0), (0, 1), (0, 2),
            (1, None), (1, 0), (1, 1), (1, 2),
            (2, None), (2, 0), (2, 1), (2, 2)]

    g = _sc_gather_all(tables, [ah, at, ar], gsel, bounds, perm)
    s_perm = _tc_score(g, bounds_tc)[:, 0]
    s = s_perm[pos]
    return (s[:4096], s[4096:])
